# baseline (speedup 1.0000x reference)
import jax
import jax.numpy as jnp
from jax import lax
from jax.experimental import pallas as pl
from jax.experimental.pallas import tpu as pltpu
from jax.experimental.pallas import tpu_sc as plsc

NUM_FIELDS = 26
FIELD_SIZE = 100000
BATCH = 16384
NUM_WORKERS = 32
B_PER_W = BATCH // NUM_WORKERS
CHUNK = 16
N_CHUNKS = B_PER_W // CHUNK
LANES = 16
PREFIX = 2599936
NVAL = NUM_FIELDS * B_PER_W
LAST = NUM_FIELDS - 1
CUT = PREFIX - LAST * FIELD_SIZE
TAIL_BASE = 2599808
TAIL_OFF = TAIL_BASE - LAST * FIELD_SIZE
TAIL_LEN = 2600000 - TAIL_BASE
HALF = 13


def _sc_kernel(xt_hbm, ta_hbm, tc_hbm, bias_hbm, out_hbm,
               idx_v, val_v, ia_v, tc_v, acc_v, bias_v, sem, semb):
    wid = lax.axis_index("s") * 2 + lax.axis_index("c")
    base = wid * B_PER_W

    idx_cp = pltpu.make_async_copy(
        xt_hbm.at[:, pl.ds(base, B_PER_W)], idx_v, sem)
    idx_cp.start()

    pltpu.sync_copy(bias_hbm, bias_v)
    pltpu.sync_copy(tc_hbm, tc_v)
    idx_cp.wait()

    @pl.loop(0, HALF)
    def _fire(f):
        tview = ta_hbm.at[pl.ds(f * FIELD_SIZE, FIELD_SIZE)]
        for q in range(N_CHUNKS):
            pltpu.async_copy(
                tview.at[idx_v.at[f, pl.ds(q * CHUNK, CHUNK)]],
                val_v.at[pl.ds(f * B_PER_W + q * CHUNK, CHUNK)],
                sem,
            )

    @pl.loop(HALF, NUM_FIELDS - 1)
    def _fire2(f):
        tview = ta_hbm.at[pl.ds(f * FIELD_SIZE, FIELD_SIZE)]
        for q in range(N_CHUNKS):
            pltpu.async_copy(
                tview.at[idx_v.at[f, pl.ds(q * CHUNK, CHUNK)]],
                val_v.at[pl.ds(f * B_PER_W + q * CHUNK, CHUNK)],
                semb,
            )

    @pl.loop(0, B_PER_W, step=LANES)
    def _prep(j):
        xi = idx_v[LAST, pl.ds(j, LANES)]
        ia_v[pl.ds(j, LANES)] = jnp.minimum(xi, CUT - 1)

    tlast = ta_hbm.at[pl.ds(LAST * FIELD_SIZE, CUT)]
    for q in range(N_CHUNKS):
        pltpu.async_copy(
            tlast.at[ia_v.at[pl.ds(q * CHUNK, CHUNK)]],
            val_v.at[pl.ds(LAST * B_PER_W + q * CHUNK, CHUNK)],
            semb,
        )

    pltpu.make_async_copy(
        ta_hbm.at[pl.ds(0, HALF * B_PER_W)], val_v.at[pl.ds(0, HALF * B_PER_W)],
        sem).wait()

    @pl.loop(0, B_PER_W, step=LANES)
    def _reduce_a(j):
        acc = jnp.full((LANES,), 0.0, jnp.float32)
        for f in range(HALF):
            acc = acc + val_v[pl.ds(f * B_PER_W + j, LANES)]
        acc_v[pl.ds(j, LANES)] = acc

    pltpu.make_async_copy(
        ta_hbm.at[pl.ds(0, (NUM_FIELDS - HALF) * B_PER_W)],
        val_v.at[pl.ds(HALF * B_PER_W, (NUM_FIELDS - HALF) * B_PER_W)],
        semb).wait()

    b = plsc.load_gather(bias_v, [jax.lax.iota(jnp.int32, 16) * 0])

    @pl.loop(0, B_PER_W, step=LANES)
    def _reduce_b(j):
        acc = acc_v[pl.ds(j, LANES)]
        for f in range(HALF, NUM_FIELDS - 1):
            acc = acc + val_v[pl.ds(f * B_PER_W + j, LANES)]
        xi = idx_v[LAST, pl.ds(j, LANES)]
        va = val_v[pl.ds(LAST * B_PER_W + j, LANES)]
        ic = jnp.maximum(xi - TAIL_OFF, 0)
        vc = plsc.load_gather(tc_v, [ic])
        acc = acc + jnp.where(xi >= CUT, vc, va)
        acc_v[pl.ds(j, LANES)] = 1.0 / (1.0 + jnp.exp(-(acc + b)))

    pltpu.sync_copy(acc_v, out_hbm.at[pl.ds(base, B_PER_W)])


@jax.jit
def kernel(x, table, bias):
    xt = x.astype(jnp.int32).T
    ta = table[:PREFIX, 0]
    tc = table[TAIL_BASE:, 0]
    mesh = plsc.VectorSubcoreMesh(core_axis_name="c", subcore_axis_name="s")
    k = pl.kernel(
        _sc_kernel,
        out_type=jax.ShapeDtypeStruct((BATCH,), jnp.float32),
        mesh=mesh,
        compiler_params=pltpu.CompilerParams(
            needs_layout_passes=False, skip_device_barrier=True),
        scratch_types=[
            pltpu.VMEM((NUM_FIELDS, B_PER_W), jnp.int32),
            pltpu.VMEM((NVAL,), jnp.float32),
            pltpu.VMEM((B_PER_W,), jnp.int32),
            pltpu.VMEM((TAIL_LEN,), jnp.float32),
            pltpu.VMEM((B_PER_W,), jnp.float32),
            pltpu.VMEM((1,), jnp.float32),
            pltpu.SemaphoreType.DMA,
            pltpu.SemaphoreType.DMA,
        ],
    )
    return k(xt, ta, tc, bias)


# speedup vs baseline: 1.0102x; 1.0102x over previous
import jax
import jax.numpy as jnp
from jax import lax
from jax.experimental import pallas as pl
from jax.experimental.pallas import tpu as pltpu
from jax.experimental.pallas import tpu_sc as plsc

NUM_FIELDS = 26
FIELD_SIZE = 100000
BATCH = 16384
NUM_WORKERS = 32
B_PER_W = BATCH // NUM_WORKERS
CHUNK = 32
N_CHUNKS = B_PER_W // CHUNK
LANES = 16
PREFIX = 2599936
NVAL = NUM_FIELDS * B_PER_W
LAST = NUM_FIELDS - 1
CUT = PREFIX - LAST * FIELD_SIZE
TAIL_BASE = 2599808
TAIL_OFF = TAIL_BASE - LAST * FIELD_SIZE
TAIL_LEN = 2600000 - TAIL_BASE
HALF = 13


def _sc_kernel(xt_hbm, ta_hbm, tc_hbm, bias_hbm, out_hbm,
               idx_v, val_v, ia_v, tc_v, acc_v, bias_v, sem, semb):
    wid = lax.axis_index("s") * 2 + lax.axis_index("c")
    base = wid * B_PER_W

    idx_cp = pltpu.make_async_copy(
        xt_hbm.at[:, pl.ds(base, B_PER_W)], idx_v, sem)
    idx_cp.start()

    pltpu.sync_copy(bias_hbm, bias_v)
    pltpu.sync_copy(tc_hbm, tc_v)
    idx_cp.wait()

    @pl.loop(0, HALF)
    def _fire(f):
        tview = ta_hbm.at[pl.ds(f * FIELD_SIZE, FIELD_SIZE)]
        for q in range(N_CHUNKS):
            pltpu.async_copy(
                tview.at[idx_v.at[f, pl.ds(q * CHUNK, CHUNK)]],
                val_v.at[pl.ds(f * B_PER_W + q * CHUNK, CHUNK)],
                sem,
            )

    @pl.loop(HALF, NUM_FIELDS - 1)
    def _fire2(f):
        tview = ta_hbm.at[pl.ds(f * FIELD_SIZE, FIELD_SIZE)]
        for q in range(N_CHUNKS):
            pltpu.async_copy(
                tview.at[idx_v.at[f, pl.ds(q * CHUNK, CHUNK)]],
                val_v.at[pl.ds(f * B_PER_W + q * CHUNK, CHUNK)],
                semb,
            )

    @pl.loop(0, B_PER_W, step=LANES)
    def _prep(j):
        xi = idx_v[LAST, pl.ds(j, LANES)]
        ia_v[pl.ds(j, LANES)] = jnp.minimum(xi, CUT - 1)

    tlast = ta_hbm.at[pl.ds(LAST * FIELD_SIZE, CUT)]
    for q in range(N_CHUNKS):
        pltpu.async_copy(
            tlast.at[ia_v.at[pl.ds(q * CHUNK, CHUNK)]],
            val_v.at[pl.ds(LAST * B_PER_W + q * CHUNK, CHUNK)],
            semb,
        )

    pltpu.make_async_copy(
        ta_hbm.at[pl.ds(0, HALF * B_PER_W)], val_v.at[pl.ds(0, HALF * B_PER_W)],
        sem).wait()

    @pl.loop(0, B_PER_W, step=LANES)
    def _reduce_a(j):
        acc = jnp.full((LANES,), 0.0, jnp.float32)
        for f in range(HALF):
            acc = acc + val_v[pl.ds(f * B_PER_W + j, LANES)]
        acc_v[pl.ds(j, LANES)] = acc

    pltpu.make_async_copy(
        ta_hbm.at[pl.ds(0, (NUM_FIELDS - HALF) * B_PER_W)],
        val_v.at[pl.ds(HALF * B_PER_W, (NUM_FIELDS - HALF) * B_PER_W)],
        semb).wait()

    b = plsc.load_gather(bias_v, [jax.lax.iota(jnp.int32, 16) * 0])

    @pl.loop(0, B_PER_W, step=LANES)
    def _reduce_b(j):
        acc = acc_v[pl.ds(j, LANES)]
        for f in range(HALF, NUM_FIELDS - 1):
            acc = acc + val_v[pl.ds(f * B_PER_W + j, LANES)]
        xi = idx_v[LAST, pl.ds(j, LANES)]
        va = val_v[pl.ds(LAST * B_PER_W + j, LANES)]
        ic = jnp.maximum(xi - TAIL_OFF, 0)
        vc = plsc.load_gather(tc_v, [ic])
        acc = acc + jnp.where(xi >= CUT, vc, va)
        acc_v[pl.ds(j, LANES)] = 1.0 / (1.0 + jnp.exp(-(acc + b)))

    pltpu.sync_copy(acc_v, out_hbm.at[pl.ds(base, B_PER_W)])


@jax.jit
def kernel(x, table, bias):
    xt = x.astype(jnp.int32).T
    ta = table[:PREFIX, 0]
    tc = table[TAIL_BASE:, 0]
    mesh = plsc.VectorSubcoreMesh(core_axis_name="c", subcore_axis_name="s")
    k = pl.kernel(
        _sc_kernel,
        out_type=jax.ShapeDtypeStruct((BATCH,), jnp.float32),
        mesh=mesh,
        compiler_params=pltpu.CompilerParams(
            needs_layout_passes=False, skip_device_barrier=True),
        scratch_types=[
            pltpu.VMEM((NUM_FIELDS, B_PER_W), jnp.int32),
            pltpu.VMEM((NVAL,), jnp.float32),
            pltpu.VMEM((B_PER_W,), jnp.int32),
            pltpu.VMEM((TAIL_LEN,), jnp.float32),
            pltpu.VMEM((B_PER_W,), jnp.float32),
            pltpu.VMEM((1,), jnp.float32),
            pltpu.SemaphoreType.DMA,
            pltpu.SemaphoreType.DMA,
        ],
    )
    return k(xt, ta, tc, bias)
